# baseline (device time: 32257 ns/iter reference)
import jax
import jax.numpy as jnp
from jax import lax
from jax.experimental import pallas as pl
from jax.experimental.pallas import tpu as pltpu

Q = 4
C = 8


def kernel(x):
    _, m, n2 = x.shape
    n = n2 // 2
    h = n // 2
    qm = m // Q
    ck = qm // C

    def body(x_ref, out_ref, g_ref, sb_ref,
             z_s, z_r, xq_s, xq_r, yq_s, yq_r, xr_s, xr_r, yr_s, yr_r):
        my_x = lax.axis_index("x")
        my_y = lax.axis_index("y")
        my_z = lax.axis_index("z")

        def rows(base, c=0):
            return pl.ds(pl.multiple_of(base + c * ck, ck), ck)

        colA = pl.ds(0, h)
        colB = pl.ds(h, h)

        q_row = (2 * my_x + my_y) * qm
        qx_row = (2 * (1 - my_x) + my_y) * qm
        qy_row = (2 * my_x + (1 - my_y)) * qm
        qxy_row = (2 * (1 - my_x) + (1 - my_y)) * qm

        my_col = pl.multiple_of(my_z * n, n)
        peer_col = pl.multiple_of((1 - my_z) * n, n)

        z_peer = (my_x, my_y, 1 - my_z)
        x_nbr = (1 - my_x, my_y, my_z)
        y_nbr = (my_x, 1 - my_y, my_z)

        def rcopy(src, dst, ssem, rsem, dev):
            return pltpu.make_async_remote_copy(
                src_ref=src, dst_ref=dst, send_sem=ssem, recv_sem=rsem,
                device_id=dev, device_id_type=pl.DeviceIdType.MESH,
            )

        def add_chunk(base, c):
            out_ref[rows(base, c), :] = (
                x_ref[0, rows(base, c), pl.ds(my_col, n)]
                + g_ref[rows(base, c), :]
            )

        barrier_sem = pltpu.get_barrier_semaphore()
        for nbr in (z_peer, x_nbr, y_nbr):
            pl.semaphore_signal(
                barrier_sem, inc=1, device_id=nbr,
                device_id_type=pl.DeviceIdType.MESH,
            )
        pl.semaphore_wait(barrier_sem, 3)

        sends = []
        for c in range(C):
            rd = rcopy(
                x_ref.at[0, rows(q_row, c), pl.ds(peer_col, n)],
                g_ref.at[rows(q_row, c), :],
                z_s.at[c], z_r.at[c], z_peer,
            )
            rd.start()
            sends.append(rd)
        for i, t_row in enumerate((qx_row, qy_row, qxy_row)):
            rd = rcopy(
                x_ref.at[0, rows(t_row, 0), pl.ds(peer_col, n)],
                g_ref.at[rows(t_row, 0), :],
                z_s.at[C + i], z_r.at[C + i], z_peer,
            )
            rd.start()
            sends.append(rd)

        z_recv = [
            rcopy(sb_ref.at[rows(0, c), :], g_ref.at[rows(q_row, c), :],
                  z_s.at[c], z_r.at[c], z_peer)
            for c in range(C)
        ]
        ze_recv = [
            rcopy(sb_ref.at[rows(0, 0), :], g_ref.at[rows(t_row, 0), :],
                  z_s.at[C + i], z_r.at[C + i], z_peer)
            for i, t_row in enumerate((qx_row, qy_row, qxy_row))
        ]
        xq_recv = {
            c: rcopy(sb_ref.at[rows(0, c), :], g_ref.at[rows(qx_row, c), :],
                     xq_s.at[c - 1], xq_r.at[c - 1], x_nbr)
            for c in range(1, C)
        }
        yq_recv = {
            c: rcopy(sb_ref.at[rows(0, c), :], g_ref.at[rows(qy_row, c), :],
                     yq_s.at[c - 1], yq_r.at[c - 1], y_nbr)
            for c in range(1, C)
        }
        yr_recv = {
            c: rcopy(sb_ref.at[rows(0, c), colA],
                     g_ref.at[rows(qxy_row, c), colA],
                     yr_s.at[c - 1], yr_r.at[c - 1], y_nbr)
            for c in range(1, C)
        }
        xr_recv = {
            c: rcopy(sb_ref.at[rows(0, c), colB],
                     g_ref.at[rows(qxy_row, c), colB],
                     xr_s.at[c - 1], xr_r.at[c - 1], x_nbr)
            for c in range(1, C)
        }

        def fwd_own(c):
            for ssem, rsem, dev in ((xq_s, xq_r, x_nbr), (yq_s, yq_r, y_nbr)):
                rd = rcopy(
                    g_ref.at[rows(q_row, c), :], g_ref.at[rows(q_row, c), :],
                    ssem.at[c - 1], rsem.at[c - 1], dev,
                )
                rd.start()
                sends.append(rd)

        def relay_qx(c):
            rd = rcopy(
                g_ref.at[rows(qx_row, c), colA], g_ref.at[rows(qx_row, c), colA],
                yr_s.at[c - 1], yr_r.at[c - 1], y_nbr,
            )
            rd.start()
            sends.append(rd)

        def relay_qy(c):
            rd = rcopy(
                g_ref.at[rows(qy_row, c), colB], g_ref.at[rows(qy_row, c), colB],
                xr_s.at[c - 1], xr_r.at[c - 1], x_nbr,
            )
            rd.start()
            sends.append(rd)

        for c in range(C):
            z_recv[c].wait_recv()
            if c >= 1:
                fwd_own(c)
            add_chunk(q_row, c)
            if c >= 2:
                cc = c - 1
                xq_recv[cc].wait_recv()
                relay_qx(cc)
                add_chunk(qx_row, cc)
                yq_recv[cc].wait_recv()
                relay_qy(cc)
                add_chunk(qy_row, cc)
        xq_recv[C - 1].wait_recv()
        relay_qx(C - 1)
        add_chunk(qx_row, C - 1)
        yq_recv[C - 1].wait_recv()
        relay_qy(C - 1)
        add_chunk(qy_row, C - 1)

        for rd, t_row in zip(ze_recv, (qx_row, qy_row, qxy_row)):
            rd.wait_recv()
            add_chunk(t_row, 0)

        for c in range(1, C):
            yr_recv[c].wait_recv()
            xr_recv[c].wait_recv()
            add_chunk(qxy_row, c)

        for rd in sends:
            rd.wait_send()

    return pl.pallas_call(
        body,
        out_shape=jax.ShapeDtypeStruct((m, n), jnp.float32),
        in_specs=[pl.BlockSpec(memory_space=pltpu.VMEM)],
        out_specs=pl.BlockSpec(memory_space=pltpu.VMEM),
        scratch_shapes=[
            pltpu.VMEM((m, n), jnp.float32),
            pltpu.VMEM((m // Q, n), jnp.float32),
            pltpu.SemaphoreType.DMA((C + 3,)),
            pltpu.SemaphoreType.DMA((C + 3,)),
            pltpu.SemaphoreType.DMA((C - 1,)),
            pltpu.SemaphoreType.DMA((C - 1,)),
            pltpu.SemaphoreType.DMA((C - 1,)),
            pltpu.SemaphoreType.DMA((C - 1,)),
            pltpu.SemaphoreType.DMA((C - 1,)),
            pltpu.SemaphoreType.DMA((C - 1,)),
            pltpu.SemaphoreType.DMA((C - 1,)),
            pltpu.SemaphoreType.DMA((C - 1,)),
        ],
        compiler_params=pltpu.CompilerParams(collective_id=0),
    )(x)
